# baseline (speedup 1.0000x reference)
import functools

import jax
import jax.numpy as jnp
from jax import lax
from jax.experimental import pallas as pl
from jax.experimental.pallas import tpu as pltpu
from jax.experimental.pallas import tpu_sc as plsc

N_PROTOS = 8192
BQ = 1024
NQ = 8192


def _tc_body(q_ref, p_ref, base_ref, idx_ref, pn_ref):
    @pl.when(pl.program_id(0) == 0)
    def _init():
        p = p_ref[...]
        ones = jnp.ones((1, p.shape[1]), jnp.float32)
        pn_ref[...] = jax.lax.dot_general(
            ones, p * p, (((1,), (1,)), ((), ())),
            preferred_element_type=jnp.float32,
        )

    q = q_ref[0]
    s = jax.lax.dot_general(
        q, p_ref[...], (((1,), (1,)), ((), ())),
        preferred_element_type=jnp.float32,
    )
    m = jnp.max(s, axis=1, keepdims=True)
    iota = jax.lax.broadcasted_iota(jnp.int32, s.shape, 1)
    idx = jnp.min(jnp.where(s == m, iota, N_PROTOS), axis=1)
    qn = jnp.sum(q * q, axis=1)
    base_ref[0, 0, :] = qn - 2.0 * m[:, 0]
    idx_ref[0, 0, :] = idx


def _tc_call(queries, prototypes):
    B, L, C = queries.shape
    n_lb = L // BQ
    grid = (B * n_lb,)
    return pl.pallas_call(
        _tc_body,
        grid=grid,
        in_specs=[
            pl.BlockSpec((1, BQ, C), lambda g: (g // n_lb, g % n_lb, 0)),
            pl.BlockSpec(prototypes.shape, lambda g: (0, 0)),
        ],
        out_specs=[
            pl.BlockSpec((1, 1, BQ), lambda g: (g, 0, 0)),
            pl.BlockSpec((1, 1, BQ), lambda g: (g, 0, 0)),
            pl.BlockSpec((1, N_PROTOS), lambda g: (0, 0)),
        ],
        out_shape=[
            jax.ShapeDtypeStruct((B * n_lb, 1, BQ), jnp.float32),
            jax.ShapeDtypeStruct((B * n_lb, 1, BQ), jnp.int32),
            jax.ShapeDtypeStruct((1, N_PROTOS), jnp.float32),
        ],
    )(queries, prototypes)


_NC, _NS, _L = 2, 16, 16
_NW = _NC * _NS
_BPW = NQ // _NW


_sc_gather_cache = []


def _get_sc_gather():
    if _sc_gather_cache:
        return _sc_gather_cache[0]

    @functools.partial(
        pl.kernel,
        mesh=plsc.VectorSubcoreMesh(
            core_axis_name="c", subcore_axis_name="s",
            num_cores=_NC, num_subcores=_NS,
        ),
        out_type=jax.ShapeDtypeStruct((NQ,), jnp.float32),
        scratch_types=[
            pltpu.VMEM((_BPW,), jnp.int32),
            pltpu.VMEM((_BPW,), jnp.float32),
            pltpu.VMEM((_BPW,), jnp.float32),
            pltpu.VMEM((_BPW,), jnp.float32),
            pltpu.SemaphoreType.DMA,
        ],
    )
    def _sc_gather(idx_hbm, base_hbm, pn_hbm, out_hbm,
                   idx_v, base_v, vals_v, out_v, sem):
        wid = lax.axis_index("s") * _NC + lax.axis_index("c")
        base_off = wid * _BPW
        pltpu.sync_copy(idx_hbm.at[pl.ds(base_off, _BPW)], idx_v)
        pltpu.sync_copy(base_hbm.at[pl.ds(base_off, _BPW)], base_v)
        pltpu.async_copy(pn_hbm.at[idx_v], vals_v, sem).wait()
        for j in range(_BPW // _L):
            sl = pl.ds(j * _L, _L)
            out_v[sl] = base_v[sl] + vals_v[sl]
        pltpu.sync_copy(out_v, out_hbm.at[pl.ds(base_off, _BPW)])

    _sc_gather_cache.append(_sc_gather)
    return _sc_gather


@jax.jit
def kernel(queries, prototypes):
    B, L, C = queries.shape
    base, idx, pn = _tc_call(queries, prototypes)
    out = _get_sc_gather()(
        idx.reshape(NQ), base.reshape(NQ), pn.reshape(N_PROTOS))
    return out.reshape(B, L)


# speedup vs baseline: 1.4483x; 1.4483x over previous
import jax
import jax.numpy as jnp
from jax.experimental import pallas as pl
from jax.experimental.pallas import tpu as pltpu

N_PROTOS = 8192
BQ = 1024


def _body(q_ref, p_ref, out_ref, pn_ref):
    @pl.when(pl.program_id(0) == 0)
    def _init():
        p = p_ref[...]
        ones = jnp.ones((1, p.shape[1]), jnp.float32)
        pn_ref[...] = jax.lax.dot_general(
            ones, p * p, (((1,), (1,)), ((), ())),
            preferred_element_type=jnp.float32,
        )

    q = q_ref[0]
    s = jax.lax.dot_general(
        q, p_ref[...], (((1,), (1,)), ((), ())),
        preferred_element_type=jnp.float32,
    )
    m = jnp.max(s, axis=1, keepdims=True)
    pn_sel = jnp.min(
        jnp.where(s == m, pn_ref[...], jnp.float32(jnp.inf)), axis=1
    )
    qn = jnp.sum(q * q, axis=1)
    out_ref[0, 0, :] = qn - 2.0 * m[:, 0] + pn_sel


@jax.jit
def kernel(queries, prototypes):
    B, L, C = queries.shape
    n_lb = L // BQ
    grid = (B * n_lb,)
    out = pl.pallas_call(
        _body,
        grid=grid,
        in_specs=[
            pl.BlockSpec((1, BQ, C), lambda g: (g // n_lb, g % n_lb, 0)),
            pl.BlockSpec(prototypes.shape, lambda g: (0, 0)),
        ],
        out_specs=pl.BlockSpec((1, 1, BQ), lambda g: (g, 0, 0)),
        out_shape=jax.ShapeDtypeStruct((B * n_lb, 1, BQ), jnp.float32),
        scratch_shapes=[pltpu.VMEM((1, N_PROTOS), jnp.float32)],
    )(queries, prototypes)
    return out.reshape(B, L)
